# baseline (speedup 1.0000x reference)
import functools

import jax
import jax.numpy as jnp
from jax import lax
from jax.experimental import pallas as pl
from jax.experimental.pallas import tpu as pltpu
from jax.experimental.pallas import tpu_sc as plsc

_B, _S, _V = 8, 2048, 4096
_TOK = _B * _S
_NC, _NS, _L = 2, 16, 16
_NW = _NC * _NS
_PW = _TOK // _NW
_CH = 32
_NST = _PW // _CH


@functools.partial(
    pl.kernel,
    mesh=plsc.VectorSubcoreMesh(core_axis_name="c", subcore_axis_name="s"),
    out_type=jax.ShapeDtypeStruct((_NW, _L), jnp.float32),
    scratch_types=[
        pltpu.VMEM((_PW,), jnp.int32),
        pltpu.VMEM((_PW,), jnp.int32),
        pltpu.VMEM((_CH * 8, 128), jnp.float32),
        pltpu.VMEM((_CH * 8, 128), jnp.float32),
        pltpu.VMEM((_L,), jnp.float32),
        pltpu.SemaphoreType.DMA,
        pltpu.SemaphoreType.DMA,
    ],
    compiler_params=pltpu.CompilerParams(needs_layout_passes=False),
)
def _nll_partials(x_hbm, gold_hbm, out_hbm, gold_v, q_v, val_a, val_b,
                  acc_v, sem0, sem1):
    wid = lax.axis_index("s") * _NC + lax.axis_index("c")
    base = pl.multiple_of(wid * _PW, _PW)
    pltpu.sync_copy(gold_hbm.at[pl.ds(base, _PW)], gold_v)

    lane = lax.iota(jnp.int32, 16)
    for c in range(_PW // _L):
        off = c * _L
        g = gold_v[pl.ds(off, _L)]
        t = base + off + lane
        q_v[pl.ds(off, _L)] = (
            lax.shift_left(lax.shift_right_logical(t, 3), 5)
            + lax.shift_right_logical(g, 7)
        )

    bufs = (val_a, val_b)
    sems = (sem0, sem1)

    def _fire(st):
        buf, sem = bufs[st % 2], sems[st % 2]
        handles = []
        for c in range(_CH // _L):
            q16 = q_v[pl.ds(st * _CH + c * _L, _L)]
            for j in range(_L):
                slot = c * _L + j
                q = q16[j]
                r8 = lax.shift_left(lax.shift_right_logical(q, 5), 3)
                k0 = lax.shift_left(lax.bitwise_and(q, 31), 7)
                src = x_hbm.at[pl.ds(pl.multiple_of(r8, 8), 8),
                               pl.ds(pl.multiple_of(k0, 128), 128)]
                cp = pltpu.async_copy(src, buf.at[pl.ds(slot * 8, 8), :], sem)
                handles.append(cp)
        return handles

    sub = lax.bitwise_and(lane, 7)

    def _consume(st, acc):
        buf = bufs[st % 2]
        for c in range(_CH // _L):
            off = st * _CH + c * _L
            sel = lax.bitwise_and(gold_v[pl.ds(off, _L)], 127)
            rows = (c * _L + lane) * 8 + sub
            acc = acc + plsc.load_gather(buf, [rows, sel])
        return acc

    acc = jnp.zeros((_L,), jnp.float32)
    pending = _fire(0)
    for st in range(_NST):
        nxt = _fire(st + 1) if st + 1 < _NST else []
        for cp in pending:
            cp.wait()
        pending = nxt
        acc = _consume(st, acc)
    acc_v[...] = acc
    pltpu.sync_copy(acc_v, out_hbm.at[wid])


def kernel(x, gold):
    x2 = x.reshape(_TOK, _V)
    gf = gold.reshape(-1).astype(jnp.int32)
    partials = _nll_partials(x2, gf)
    return -(jnp.sum(partials) / jnp.float32(_B))


# speedup vs baseline: 2.4171x; 2.4171x over previous
import functools

import jax
import jax.numpy as jnp
from jax import lax
from jax.experimental import pallas as pl
from jax.experimental.pallas import tpu as pltpu
from jax.experimental.pallas import tpu_sc as plsc

_B, _S, _V = 8, 2048, 4096
_TOK = _B * _S
_NC, _NS, _L = 2, 16, 16
_NW = _NC * _NS
_PW = _TOK // _NW
_G = 128


@functools.partial(
    pl.kernel,
    mesh=plsc.VectorSubcoreMesh(core_axis_name="c", subcore_axis_name="s"),
    out_type=jax.ShapeDtypeStruct((_NW, _L), jnp.float32),
    scratch_types=[
        pltpu.VMEM((_PW,), jnp.int32),
        pltpu.VMEM((_PW,), jnp.int32),
        pltpu.VMEM((_PW,), jnp.float32),
        pltpu.VMEM((_L,), jnp.float32),
        pltpu.SemaphoreType.DMA,
    ],
)
def _nll_partials(x_hbm, gold_hbm, out_hbm, gold_v, idx_v, val_v, acc_v,
                  sem):
    wid = lax.axis_index("s") * _NC + lax.axis_index("c")
    base = pl.multiple_of(wid * _PW, _PW)
    pltpu.sync_copy(gold_hbm.at[pl.ds(base, _PW)], gold_v)

    lane = lax.iota(jnp.int32, _L)
    for c in range(_PW // _L):
        off = c * _L
        g = gold_v[pl.ds(off, _L)]
        t = base + off + lane
        idx_v[pl.ds(off, _L)] = (
            lax.shift_left(lax.shift_right_logical(t, 3), 15)
            + lax.shift_left(lax.shift_right_logical(g, 7), 10)
            + lax.shift_left(lax.bitwise_and(t, 7), 7)
            + lax.bitwise_and(g, 127)
        )

    copies = [
        pltpu.async_copy(
            x_hbm.at[idx_v.at[pl.ds(k * _G, _G)]],
            val_v.at[pl.ds(k * _G, _G)],
            sem,
        )
        for k in range(_PW // _G)
    ]
    for cp in copies:
        cp.wait()

    acc = jnp.zeros((_L,), jnp.float32)
    for c in range(_PW // _L):
        acc = acc + val_v[pl.ds(c * _L, _L)]
    acc_v[...] = acc
    pltpu.sync_copy(acc_v, out_hbm.at[wid])


def kernel(x, gold):
    xt = x.reshape(2048, 8, 32, 128).transpose(0, 2, 1, 3).reshape(-1)
    gf = gold.reshape(-1).astype(jnp.int32)
    partials = _nll_partials(xt, gf)
    return -(jnp.sum(partials) / jnp.float32(_B))


# speedup vs baseline: 2.4560x; 1.0161x over previous
import functools

import jax
import jax.numpy as jnp
from jax import lax
from jax.experimental import pallas as pl
from jax.experimental.pallas import tpu as pltpu
from jax.experimental.pallas import tpu_sc as plsc

_B, _S, _V = 8, 2048, 4096
_TOK = _B * _S
_NC, _NS, _L = 2, 16, 16
_NW = _NC * _NS
_PW = _TOK // _NW
_G = 128


@functools.partial(
    pl.kernel,
    mesh=plsc.VectorSubcoreMesh(core_axis_name="c", subcore_axis_name="s"),
    out_type=jax.ShapeDtypeStruct((_NW, _L), jnp.float32),
    scratch_types=[
        pltpu.VMEM((_PW,), jnp.int32),
        pltpu.VMEM((_PW,), jnp.int32),
        pltpu.VMEM((_PW,), jnp.float32),
        pltpu.VMEM((_L,), jnp.float32),
        pltpu.SemaphoreType.DMA,
        pltpu.SemaphoreType.DMA,
        pltpu.SemaphoreType.DMA,
        pltpu.SemaphoreType.DMA,
    ],
)
def _nll_partials(x_hbm, gold_hbm, out_hbm, gold_v, idx_v, val_v, acc_v,
                  *sems):
    wid = lax.axis_index("s") * _NC + lax.axis_index("c")
    base = pl.multiple_of(wid * _PW, _PW)
    pltpu.sync_copy(gold_hbm.at[pl.ds(base, _PW)], gold_v)

    lane = lax.iota(jnp.int32, _L)
    tvec = (
        lax.shift_left(lax.shift_right_logical(lane, 3), 15)
        + lax.shift_left(lax.bitwise_and(lane, 7), 7)
    )

    copies = []
    for k in range(_PW // _G):
        for c in range(k * (_G // _L), (k + 1) * (_G // _L)):
            off = c * _L
            g = gold_v[pl.ds(off, _L)]
            idx_v[pl.ds(off, _L)] = (
                (tvec + lax.shift_left(base + off, 12))
                + lax.shift_left(lax.shift_right_logical(g, 7), 10)
                + lax.bitwise_and(g, 127)
            )
        copies.append(
            pltpu.async_copy(
                x_hbm.at[idx_v.at[pl.ds(k * _G, _G)]],
                val_v.at[pl.ds(k * _G, _G)],
                sems[k],
            )
        )

    acc = jnp.zeros((_L,), jnp.float32)
    for k, cp in enumerate(copies):
        cp.wait()
        for c in range(k * (_G // _L), (k + 1) * (_G // _L)):
            acc = acc + val_v[pl.ds(c * _L, _L)]
    acc_v[...] = acc
    pltpu.sync_copy(acc_v, out_hbm.at[wid])


def kernel(x, gold):
    xt = x.reshape(2048, 8, 32, 128).transpose(0, 2, 1, 3).reshape(-1)
    gf = gold.reshape(-1).astype(jnp.int32)
    partials = _nll_partials(xt, gf)
    return -(jnp.sum(partials) / jnp.float32(_B))
